# Initial kernel scaffold; baseline (speedup 1.0000x reference)
import jax
import jax.numpy as jnp
from jax.experimental import pallas as pl


def kernel(x_user, x_item, edge_u2i, edge_i2u, edge_u2u, Wl_0_u2i, bl_0_u2i, Wr_0_u2i, Wl_0_i2u, bl_0_i2u, Wr_0_i2u, Wl_0_u2u, bl_0_u2u, Wr_0_u2u, Wl_1_u2i, bl_1_u2i, Wr_1_u2i, Wl_1_i2u, bl_1_i2u, Wr_1_i2u, Wl_1_u2u, bl_1_u2u, Wr_1_u2u, lin_W, lin_b):
    raise NotImplementedError("write your pallas kernel here")


# speedup vs baseline: 2.6900x; 2.6900x over previous
import functools

import jax
import jax.numpy as jnp
from jax import lax
from jax.experimental import pallas as pl
from jax.experimental.pallas import tpu as pltpu
from jax.experimental.pallas import tpu_sc as plsc

N = 10000
D = 256
H = 128
E = 160000
OUT = 64
NSUB = 16
EPS = E // NSUB
CH = 80
NCHUNK = EPS // CH
N_PAD = 10240
RPS = N_PAD // NSUB
ZROWS = 32
CNT_PAD = 10240
CNT_PS = CNT_PAD // NSUB
BN = 1000
GRID = N // BN


def _make_sc_agg(with_counts):
    out_types = [jax.ShapeDtypeStruct((N_PAD, H), jnp.float32)
                 for _ in range(6)]
    if with_counts:
        out_types += [jax.ShapeDtypeStruct((CNT_PAD,), jnp.float32)
                      for _ in range(3)]

    scratch = [
        pltpu.VMEM((CH,), jnp.int32),
        pltpu.VMEM((CH,), jnp.int32),
        pltpu.VMEM((CH, H), jnp.float32),
        pltpu.VMEM((ZROWS, H), jnp.float32),
        pltpu.VMEM_SHARED((N_PAD, H), jnp.float32),
        pltpu.SemaphoreType.DMA,
    ]
    if with_counts:
        scratch += [
            pltpu.VMEM((CH,), jnp.float32),
            pltpu.VMEM((CNT_PS,), jnp.float32),
            pltpu.VMEM_SHARED((CNT_PAD,), jnp.float32),
            pltpu.VMEM_SHARED((CNT_PAD,), jnp.float32),
            pltpu.VMEM_SHARED((CNT_PAD,), jnp.float32),
        ]

    mesh = plsc.VectorSubcoreMesh(core_axis_name="c", subcore_axis_name="s")

    @functools.partial(pl.kernel, out_type=tuple(out_types), mesh=mesh,
                       scratch_types=scratch)
    def sc_agg(xu_lo, xu_hi, xi_lo, xi_hi,
               s_u2i, d_u2i, s_i2u, d_i2u, s_u2u, d_u2u,
               o_u2i_lo, o_u2i_hi, o_i2u_lo, o_i2u_hi, o_u2u_lo, o_u2u_hi,
               *rest):
        if with_counts:
            (o_cnt0, o_cnt1, o_cnt2, src_v, dst_v, rows_v, zrow_v, acc, sem,
             ones_v, zcnt_v, cnt0, cnt1, cnt2) = rest
            cnts = [cnt0, cnt1, cnt2]
            o_cnts = [o_cnt0, o_cnt1, o_cnt2]
        else:
            (src_v, dst_v, rows_v, zrow_v, acc, sem) = rest

        cid = lax.axis_index("c")
        sid = lax.axis_index("s")
        z16 = jnp.zeros((16,), jnp.float32)

        def zr_body(i, c):
            for j in range(H // 16):
                zrow_v[i, pl.ds(j * 16, 16)] = z16
            return c
        lax.fori_loop(0, ZROWS, zr_body, 0)

        if with_counts:
            o16 = jnp.full((16,), 1.0, jnp.float32)
            for j in range(CH // 16):
                ones_v[pl.ds(j * 16, 16)] = o16

            def zc_body(i, c):
                zcnt_v[pl.ds(i * 16, 16)] = z16
                return c
            lax.fori_loop(0, CNT_PS // 16, zc_body, 0)

        srcs = [s_u2i, s_i2u, s_u2u]
        dsts = [d_u2i, d_i2u, d_u2u]
        tabs = [(xu_lo, xu_hi), (xi_lo, xi_hi), (xu_lo, xu_hi)]
        outs = [(o_u2i_lo, o_u2i_hi), (o_i2u_lo, o_i2u_hi),
                (o_u2u_lo, o_u2u_hi)]

        for et in range(3):
            for k in range(RPS // ZROWS):
                pltpu.sync_copy(
                    zrow_v, acc.at[pl.ds(sid * RPS + k * ZROWS, ZROWS)])
            if with_counts:
                @pl.when(cid == 0)
                def _():
                    pltpu.sync_copy(
                        zcnt_v, cnts[et].at[pl.ds(sid * CNT_PS, CNT_PS)])
            plsc.subcore_barrier()

            def make_chunk(tab, count):
                src_h, dst_h = srcs[et], dsts[et]

                def chunk(j, c):
                    pltpu.sync_copy(src_h.at[sid, j], src_v)
                    pltpu.sync_copy(dst_h.at[sid, j], dst_v)
                    pltpu.async_copy(tab.at[src_v], rows_v, sem).wait()
                    pltpu.sync_copy(rows_v, acc.at[dst_v], add=True)
                    if count:
                        pltpu.sync_copy(ones_v, cnts[et].at[dst_v],
                                        add=True)
                    return c
                return chunk

            @pl.when(cid == 0)
            def _():
                lax.fori_loop(0, NCHUNK,
                              make_chunk(tabs[et][0], with_counts), 0)

            @pl.when(cid == 1)
            def _():
                lax.fori_loop(0, NCHUNK, make_chunk(tabs[et][1], False), 0)

            plsc.subcore_barrier()

            @pl.when(cid == 0)
            def _():
                pltpu.sync_copy(acc.at[pl.ds(sid * RPS, RPS)],
                                outs[et][0].at[pl.ds(sid * RPS, RPS)])

            @pl.when(cid == 1)
            def _():
                pltpu.sync_copy(acc.at[pl.ds(sid * RPS, RPS)],
                                outs[et][1].at[pl.ds(sid * RPS, RPS)])

            if with_counts:
                @pl.when(cid == 0)
                def _():
                    pltpu.sync_copy(
                        cnts[et].at[pl.ds(sid * CNT_PS, CNT_PS)],
                        o_cnts[et].at[pl.ds(sid * CNT_PS, CNT_PS)])
            plsc.subcore_barrier()

    return sc_agg


_sc_agg_l0 = _make_sc_agg(True)
_sc_agg_l1 = _make_sc_agg(False)


def _row_spec():
    return pl.BlockSpec((BN, H), lambda i: (i, 0))


def _full_spec(shape):
    nd = len(shape)
    return pl.BlockSpec(shape, (lambda i: (0,) * nd))


def _cnt_spec():
    return pl.BlockSpec((BN, 1), lambda i: (i, 0))


def _dot(a, w):
    return jnp.dot(a, w, preferred_element_type=jnp.float32)


def _tc_layer0_body(su2i_lo, su2i_hi, si2u_lo, si2u_hi, su2u_lo, su2u_hi,
                    xu_lo, xu_hi, xi_lo, xi_hi,
                    c_u2i, c_i2u, c_u2u,
                    wl_u2i, wr_u2i, wl_i2u, wl_u2u, wr_user,
                    b_item, b_user,
                    o_user, o_item, o_xru_lo, o_xru_hi, o_xri_lo, o_xri_hi):
    k_u2i = jnp.maximum(c_u2i[...], 1.0)
    k_i2u = jnp.maximum(c_i2u[...], 1.0)
    k_u2u = jnp.maximum(c_u2u[...], 1.0)
    wlA = wl_u2i[...]
    wrA = wr_u2i[...]
    item = (_dot(su2i_lo[...] / k_u2i, wlA[:H])
            + _dot(su2i_hi[...] / k_u2i, wlA[H:])
            + _dot(xi_lo[...], wrA[:H])
            + _dot(xi_hi[...], wrA[H:])
            + b_item[...])
    wlB = wl_i2u[...]
    wlC = wl_u2u[...]
    wrU = wr_user[...]
    user = (_dot(si2u_lo[...] / k_i2u, wlB[:H])
            + _dot(si2u_hi[...] / k_i2u, wlB[H:])
            + _dot(su2u_lo[...] / k_u2u, wlC[:H])
            + _dot(su2u_hi[...] / k_u2u, wlC[H:])
            + _dot(xu_lo[...], wrU[:H])
            + _dot(xu_hi[...], wrU[H:])
            + b_user[...])
    o_user[...] = user
    o_item[...] = item
    xru = jnp.maximum(user, 0.0)
    xri = jnp.maximum(item, 0.0)
    o_xru_lo[...] = xru[:, :H]
    o_xru_hi[...] = xru[:, H:]
    o_xri_lo[...] = xri[:, :H]
    o_xri_hi[...] = xri[:, H:]


def _tc_layer1_body(su2i_lo, su2i_hi, si2u_lo, si2u_hi, su2u_lo, su2u_hi,
                    xu_lo, xu_hi, xi_lo, xi_hi,
                    c_u2i, c_i2u, c_u2u,
                    wl_u2i, wr_u2i, wl_i2u, wl_u2u, wr_user,
                    b_item, b_user, lin_w, lin_b,
                    o_user, o_item, o_logits):
    k_u2i = jnp.maximum(c_u2i[...], 1.0)
    k_i2u = jnp.maximum(c_i2u[...], 1.0)
    k_u2u = jnp.maximum(c_u2u[...], 1.0)
    wlA = wl_u2i[...]
    wrA = wr_u2i[...]
    item = (_dot(su2i_lo[...] / k_u2i, wlA[:H])
            + _dot(su2i_hi[...] / k_u2i, wlA[H:])
            + _dot(xi_lo[...], wrA[:H])
            + _dot(xi_hi[...], wrA[H:])
            + b_item[...])
    wlB = wl_i2u[...]
    wlC = wl_u2u[...]
    wrU = wr_user[...]
    user = (_dot(si2u_lo[...] / k_i2u, wlB[:H])
            + _dot(si2u_hi[...] / k_i2u, wlB[H:])
            + _dot(su2u_lo[...] / k_u2u, wlC[:H])
            + _dot(su2u_hi[...] / k_u2u, wlC[H:])
            + _dot(xu_lo[...], wrU[:H])
            + _dot(xu_hi[...], wrU[H:])
            + b_user[...])
    o_user[...] = user
    o_item[...] = item
    o_logits[...] = _dot(user, lin_w[...]) + lin_b[...]


def _tc_layer0(args):
    in_specs = ([_row_spec() for _ in range(10)]
                + [_cnt_spec() for _ in range(3)]
                + [_full_spec((D, D)) for _ in range(5)]
                + [_full_spec((1, D)) for _ in range(2)])
    out_specs = [pl.BlockSpec((BN, D), lambda i: (i, 0)) for _ in range(2)] \
        + [_row_spec() for _ in range(4)]
    out_shapes = [jax.ShapeDtypeStruct((N, D), jnp.float32) for _ in range(2)] \
        + [jax.ShapeDtypeStruct((N, H), jnp.float32) for _ in range(4)]
    return pl.pallas_call(
        _tc_layer0_body,
        grid=(GRID,),
        in_specs=in_specs,
        out_specs=out_specs,
        out_shape=out_shapes,
    )(*args)


def _tc_layer1(args):
    in_specs = ([_row_spec() for _ in range(10)]
                + [_cnt_spec() for _ in range(3)]
                + [_full_spec((D, D)) for _ in range(5)]
                + [_full_spec((1, D)) for _ in range(2)]
                + [_full_spec((D, OUT)), _full_spec((1, OUT))])
    out_specs = [pl.BlockSpec((BN, D), lambda i: (i, 0)) for _ in range(2)] \
        + [pl.BlockSpec((BN, OUT), lambda i: (i, 0))]
    out_shapes = [jax.ShapeDtypeStruct((N, D), jnp.float32) for _ in range(2)] \
        + [jax.ShapeDtypeStruct((N, OUT), jnp.float32)]
    return pl.pallas_call(
        _tc_layer1_body,
        grid=(GRID,),
        in_specs=in_specs,
        out_specs=out_specs,
        out_shape=out_shapes,
    )(*args)


def kernel(x_user, x_item, edge_u2i, edge_i2u, edge_u2u,
           Wl_0_u2i, bl_0_u2i, Wr_0_u2i,
           Wl_0_i2u, bl_0_i2u, Wr_0_i2u,
           Wl_0_u2u, bl_0_u2u, Wr_0_u2u,
           Wl_1_u2i, bl_1_u2i, Wr_1_u2i,
           Wl_1_i2u, bl_1_i2u, Wr_1_i2u,
           Wl_1_u2u, bl_1_u2u, Wr_1_u2u,
           lin_W, lin_b):
    f32 = jnp.float32
    xu = x_user.astype(f32)
    xi = x_item.astype(f32)
    xu_lo, xu_hi = xu[:, :H], xu[:, H:]
    xi_lo, xi_hi = xi[:, :H], xi[:, H:]

    def prep(e):
        e = e.astype(jnp.int32)
        return (e[0].reshape(NSUB, NCHUNK, CH),
                e[1].reshape(NSUB, NCHUNK, CH))

    s_u2i, d_u2i = prep(edge_u2i)
    s_i2u, d_i2u = prep(edge_i2u)
    s_u2u, d_u2u = prep(edge_u2u)

    (a_u2i_lo, a_u2i_hi, a_i2u_lo, a_i2u_hi, a_u2u_lo, a_u2u_hi,
     cnt0, cnt1, cnt2) = _sc_agg_l0(xu_lo, xu_hi, xi_lo, xi_hi,
                                    s_u2i, d_u2i, s_i2u, d_i2u, s_u2u, d_u2u)

    c_u2i = cnt0[:N][:, None]
    c_i2u = cnt1[:N][:, None]
    c_u2u = cnt2[:N][:, None]

    w0 = (Wl_0_u2i.T, Wr_0_u2i.T, Wl_0_i2u.T, Wl_0_u2u.T,
          (Wr_0_i2u + Wr_0_u2u).T,
          bl_0_u2i[None, :], (bl_0_i2u + bl_0_u2u)[None, :])
    (user0, item0, xru_lo, xru_hi, xri_lo, xri_hi) = _tc_layer0(
        (a_u2i_lo, a_u2i_hi, a_i2u_lo, a_i2u_hi, a_u2u_lo, a_u2u_hi,
         xu_lo, xu_hi, xi_lo, xi_hi, c_u2i, c_i2u, c_u2u) + w0)

    (b_u2i_lo, b_u2i_hi, b_i2u_lo, b_i2u_hi, b_u2u_lo, b_u2u_hi) = _sc_agg_l1(
        xru_lo, xru_hi, xri_lo, xri_hi,
        s_u2i, d_u2i, s_i2u, d_i2u, s_u2u, d_u2u)

    w1 = (Wl_1_u2i.T, Wr_1_u2i.T, Wl_1_i2u.T, Wl_1_u2u.T,
          (Wr_1_i2u + Wr_1_u2u).T,
          bl_1_u2i[None, :], (bl_1_i2u + bl_1_u2u)[None, :],
          lin_W.T, lin_b[None, :])
    (user1, item1, logits) = _tc_layer1(
        (b_u2i_lo, b_u2i_hi, b_i2u_lo, b_i2u_hi, b_u2u_lo, b_u2u_hi,
         xru_lo, xru_hi, xri_lo, xri_hi, c_u2i, c_i2u, c_u2u) + w1)

    return (logits, user0, item0, user1, item1)
